# Initial kernel scaffold; baseline (speedup 1.0000x reference)
import jax
import jax.numpy as jnp
from jax.experimental import pallas as pl


def kernel(logits, labels):
    raise NotImplementedError("write your pallas kernel here")


# speedup vs baseline: 14.7328x; 14.7328x over previous
import functools

import jax
import jax.numpy as jnp
from jax.experimental import pallas as pl

ALPHA = 0.75
GAMMA = 2.0
NEG_RATIO = 3
NEG_PER_NEG_SLICE = 10

_BLOCK_ROWS = 8
_MAX_FINITE_BITS = 0x7F7FFFFF


def _focal_body(logits_ref, labels_ref, num_ref, den_ref):
    l = logits_ref[...]
    y = labels_ref[...]
    valid = (y >= 0.0).astype(jnp.float32)
    t = jnp.clip(y, 0.0, None)
    bce = jnp.maximum(l, 0.0) - l * t + jnp.log1p(jnp.exp(-jnp.abs(l)))
    p = jax.nn.sigmoid(l)
    pt = t * p + (1.0 - t) * (1.0 - p)
    one_m_pt = 1.0 - pt
    alpha_w = t * ALPHA + (1.0 - t) * (1.0 - ALPHA)
    pel = alpha_w * one_m_pt * one_m_pt * bce * valid

    pos = y == 1.0
    neg = y == 0.0
    pos_sum = jnp.sum(jnp.where(pos, pel, 0.0), axis=1, keepdims=True)
    pos_cnt = jnp.sum(jnp.where(pos & (pel > 0.0), 1.0, 0.0), axis=1, keepdims=True)
    n_pos = jnp.sum(jnp.where(pos, 1, 0).astype(jnp.int32), axis=1, keepdims=True)
    n_keep = jnp.where(
        n_pos > 0,
        jnp.maximum(1, n_pos * jnp.int32(NEG_RATIO)),
        jnp.int32(NEG_PER_NEG_SLICE),
    )

    score = jnp.where(neg & (pel > 0.0), pel, -1.0)
    si = jax.lax.bitcast_convert_type(score, jnp.int32)
    cnt_posneg = jnp.sum(jnp.where(si > 0, 1, 0).astype(jnp.int32), axis=1, keepdims=True)
    k = jnp.minimum(n_keep, cnt_posneg)

    def step(_, carry):
        lo, hi = carry
        mid = lo + (hi - lo + 1) // 2
        cnt = jnp.sum(jnp.where(si >= mid, 1, 0).astype(jnp.int32), axis=1, keepdims=True)
        ge = cnt >= k
        return jnp.where(ge, mid, lo), jnp.where(ge, hi, mid - 1)

    lo0 = jnp.zeros_like(k)
    hi0 = jnp.full_like(k, _MAX_FINITE_BITS)
    tb, _ = jax.lax.fori_loop(0, 31, step, (lo0, hi0))
    tv = jax.lax.bitcast_convert_type(tb, jnp.float32)

    gt = si > tb
    s_gt = jnp.sum(jnp.where(gt, score, 0.0), axis=1, keepdims=True)
    c_gt = jnp.sum(jnp.where(gt, 1, 0).astype(jnp.int32), axis=1, keepdims=True)
    have = k > 0
    kept_sum = jnp.where(have, s_gt + (k - c_gt).astype(jnp.float32) * tv, 0.0)
    kept_cnt = jnp.where(have, k, 0).astype(jnp.float32)

    num = pos_sum + kept_sum
    den = pos_cnt + kept_cnt

    @pl.when(pl.program_id(0) == 0)
    def _init():
        num_ref[...] = jnp.zeros_like(num_ref)
        den_ref[...] = jnp.zeros_like(den_ref)

    num_ref[...] += num
    den_ref[...] += den


def kernel(logits, labels):
    B, N = logits.shape
    logits = logits.astype(jnp.float32)
    labels = labels.astype(jnp.float32)
    grid = B // _BLOCK_ROWS
    num, den = pl.pallas_call(
        _focal_body,
        grid=(grid,),
        in_specs=[
            pl.BlockSpec((_BLOCK_ROWS, N), lambda i: (i, 0)),
            pl.BlockSpec((_BLOCK_ROWS, N), lambda i: (i, 0)),
        ],
        out_specs=[
            pl.BlockSpec((_BLOCK_ROWS, 1), lambda i: (0, 0)),
            pl.BlockSpec((_BLOCK_ROWS, 1), lambda i: (0, 0)),
        ],
        out_shape=[
            jax.ShapeDtypeStruct((_BLOCK_ROWS, 1), jnp.float32),
            jax.ShapeDtypeStruct((_BLOCK_ROWS, 1), jnp.float32),
        ],
    )(logits, labels)
    total = jnp.sum(num)
    n_valid = jnp.maximum(jnp.sum(den), 1.0)
    return total / n_valid
